# baseline (speedup 1.0000x reference)
import functools

import jax
import jax.numpy as jnp
from jax.experimental import pallas as pl

H = 16
S = 2048
D = 128
M = 32768
K_NN = 16

QB = 256
MB = 2048


def _sims_body(q_ref, km_ref, o_ref):
    q = q_ref[0]
    nrm = jnp.sqrt(jnp.sum(q * q, axis=1, keepdims=True))
    qn = q / jnp.maximum(nrm, 1e-12)
    o_ref[0] = jax.lax.dot_general(
        qn, km_ref[0], (((1,), (1,)), ((), ())),
        preferred_element_type=jnp.float32)


@jax.jit
def _sims(q3, km):
    return pl.pallas_call(
        _sims_body,
        grid=(H, M // MB, S // QB),
        in_specs=[
            pl.BlockSpec((1, QB, D), lambda h, im, iq: (h, iq, 0)),
            pl.BlockSpec((1, MB, D), lambda h, im, iq: (h, im, 0)),
        ],
        out_specs=pl.BlockSpec((1, QB, MB), lambda h, im, iq: (h, iq, im)),
        out_shape=jax.ShapeDtypeStruct((H, S, M), jnp.float32),
    )(q3, km)


def kernel(inputs, query, key, value, outputs, gate, key_memories, value_memories):
    b, h, s, d = query.shape
    q3 = jnp.transpose(query, (1, 0, 2, 3)).reshape(h, b * s, d)
    sims = _sims(q3, key_memories)
    scores, ix = jax.lax.top_k(sims, K_NN)
    mv = jax.vmap(lambda vm, i: vm[i])(value_memories, ix)
    weighted = (mv * scores[..., None]).sum(axis=2)
    weighted = weighted.reshape(b, h, s, d)
    g = jax.nn.sigmoid(gate).reshape(1, h, 1, 1)
    return g * weighted + (1.0 - g) * outputs


# speedup vs baseline: 40.1738x; 40.1738x over previous
import functools

import jax
import jax.numpy as jnp
from jax.experimental import pallas as pl

H = 16
S = 2048
D = 128
M = 32768
K_NN = 16

QB = 256
MB = 2048


def _sims_body(q_ref, km_ref, o_ref):
    q = q_ref[0]
    nrm = jnp.sqrt(jnp.sum(q * q, axis=1, keepdims=True))
    qn = q / jnp.maximum(nrm, 1e-12)
    o_ref[0] = jax.lax.dot_general(
        qn, km_ref[0], (((1,), (1,)), ((), ())),
        preferred_element_type=jnp.float32)


@jax.jit
def _sims(q3, km):
    return pl.pallas_call(
        _sims_body,
        grid=(H, M // MB, S // QB),
        in_specs=[
            pl.BlockSpec((1, QB, D), lambda h, im, iq: (h, iq, 0)),
            pl.BlockSpec((1, MB, D), lambda h, im, iq: (h, im, 0)),
        ],
        out_specs=pl.BlockSpec((1, QB, MB), lambda h, im, iq: (h, iq, im)),
        out_shape=jax.ShapeDtypeStruct((H, S, M), jnp.float32),
    )(q3, km)


QT = 64


def _topk_body(sims_ref, vm_ref, od_ref, g_ref, out_ref):
    s = sims_ref[0]
    t = jnp.full((QT, 1), jnp.inf, dtype=jnp.float32)
    for _ in range(K_NN):
        t = jnp.max(jnp.where(s < t, s, -jnp.inf), axis=1, keepdims=True)
    sw = jnp.where(s >= t, s, 0.0)
    w = jax.lax.dot_general(
        sw, vm_ref[0], (((1,), (0,)), ((), ())),
        preferred_element_type=jnp.float32)
    g = g_ref[0, 0, 0]
    out_ref[0] = g * w + (1.0 - g) * od_ref[0]


@jax.jit
def _topk_weighted(sims, vm, outs3, gsig):
    return pl.pallas_call(
        _topk_body,
        grid=(H, S // QT),
        in_specs=[
            pl.BlockSpec((1, QT, M), lambda h, iq: (h, iq, 0)),
            pl.BlockSpec((1, M, D), lambda h, iq: (h, 0, 0)),
            pl.BlockSpec((1, QT, D), lambda h, iq: (h, iq, 0)),
            pl.BlockSpec((1, 1, 1), lambda h, iq: (h, 0, 0)),
        ],
        out_specs=pl.BlockSpec((1, QT, D), lambda h, iq: (h, iq, 0)),
        out_shape=jax.ShapeDtypeStruct((H, S, D), jnp.float32),
    )(sims, vm, outs3, gsig)


def kernel(inputs, query, key, value, outputs, gate, key_memories, value_memories):
    b, h, s, d = query.shape
    q3 = jnp.transpose(query, (1, 0, 2, 3)).reshape(h, b * s, d)
    sims = _sims(q3, key_memories)
    gsig = jax.nn.sigmoid(gate).reshape(h, 1, 1)
    outs3 = outputs.reshape(h, s, d)
    res = _topk_weighted(sims, value_memories, outs3, gsig)
    return res.reshape(b, h, s, d)
